# Initial kernel scaffold; baseline (speedup 1.0000x reference)
import jax
import jax.numpy as jnp
from jax.experimental import pallas as pl


def kernel(x, edge_index, W1, b1, W2, b2):
    raise NotImplementedError("write your pallas kernel here")


# speedup vs baseline: 30.1979x; 30.1979x over previous
import functools

import jax
import jax.numpy as jnp
from jax import lax
from jax.experimental import pallas as pl
from jax.experimental.pallas import tpu as pltpu
from jax.experimental.pallas import tpu_sc as plsc

N = 100000
IN_CH = 128
HID = 32
OUT = 16

LANE = 16
ROW = 128
CH = 8

NSUB = 16
N_PAD = 102400
SLAB = N_PAD // NSUB
TRASH = N
NTRASH = 1024

E = 1600000
E_PAD = 1638400
R2D = E_PAD // ROW

ZR = 400

_mesh = plsc.VectorSubcoreMesh(core_axis_name="c", subcore_axis_name="s")


def _fill_zero_2d(zbuf):
    def body(i, _):
        zbuf[i, :] = jnp.zeros((LANE,), jnp.float32)
        return 0
    lax.fori_loop(0, ZR, body, 0)


@functools.partial(
    pl.kernel,
    mesh=_mesh,
    compiler_params=pltpu.CompilerParams(use_tc_tiling_on_sc=False),
    out_type=jax.ShapeDtypeStruct((2, N_PAD), jnp.float32),
    scratch_types=[
        pltpu.VMEM((CH, ROW), jnp.int32),
        pltpu.VMEM((ROW,), jnp.float32),
        pltpu.VMEM((SLAB,), jnp.float32),
        pltpu.VMEM_SHARED((N_PAD,), jnp.float32),
        pltpu.SemaphoreType.DMA,
    ],
)
def _deg_pass(dst_hbm, out_hbm, didx, ones_v, zbuf, acc, sem):
    c = lax.axis_index("c")
    s = lax.axis_index("s")
    w = c * NSUB + s

    for i in range(ROW // LANE):
        ones_v[pl.ds(i * LANE, LANE)] = jnp.ones((LANE,), jnp.float32)

    def fz(i, _):
        zbuf[pl.ds(i * LANE, LANE)] = jnp.zeros((LANE,), jnp.float32)
        return 0
    lax.fori_loop(0, SLAB // LANE, fz, 0)
    pltpu.sync_copy(zbuf, acc.at[pl.ds(s * SLAB, SLAB)])
    plsc.subcore_barrier()

    rows_per_w = R2D // 32

    def chunk(k, _):
        r0 = w * rows_per_w + k * CH
        pltpu.sync_copy(dst_hbm.at[pl.ds(r0, CH)], didx)
        hs = [pltpu.async_copy(ones_v, acc.at[didx.at[j]], sem, add=True)
              for j in range(CH)]
        for h in hs:
            h.wait()
        return 0
    lax.fori_loop(0, rows_per_w // CH, chunk, 0)

    plsc.subcore_barrier()
    pltpu.sync_copy(acc.at[pl.ds(s * SLAB, SLAB)],
                    out_hbm.at[c, pl.ds(s * SLAB, SLAB)])


@functools.partial(
    pl.kernel,
    mesh=_mesh,
    compiler_params=pltpu.CompilerParams(use_tc_tiling_on_sc=False),
    out_type=jax.ShapeDtypeStruct((2, N_PAD, OUT), jnp.float32),
    scratch_types=[
        pltpu.VMEM((CH, ROW), jnp.int32),
        pltpu.VMEM((CH, ROW), jnp.int32),
        pltpu.VMEM((CH, ROW, OUT), jnp.float32),
        pltpu.VMEM((ZR, LANE), jnp.float32),
        pltpu.VMEM_SHARED((N_PAD, OUT), jnp.float32),
        pltpu.SemaphoreType.DMA,
        pltpu.SemaphoreType.DMA,
    ],
)
def _l1_pass(src_hbm, dst_hbm, ha_hbm, hb_hbm, out_hbm,
             sidx, didx, rows, zbuf, acc, gsem, ssem):
    c = lax.axis_index("c")
    s = lax.axis_index("s")

    _fill_zero_2d(zbuf)

    def zc(i, _):
        pltpu.sync_copy(zbuf, acc.at[pl.ds(s * SLAB + i * ZR, ZR)])
        return 0
    lax.fori_loop(0, SLAB // ZR, zc, 0)
    plsc.subcore_barrier()

    rows_per_sub = R2D // NSUB

    def run(h_hbm):
        def chunk(k, _):
            r0 = s * rows_per_sub + k * CH
            pltpu.sync_copy(src_hbm.at[pl.ds(r0, CH)], sidx)
            pltpu.sync_copy(dst_hbm.at[pl.ds(r0, CH)], didx)
            gh = [pltpu.async_copy(h_hbm.at[sidx.at[j]], rows.at[j], gsem)
                  for j in range(CH)]
            for h in gh:
                h.wait()
            sh = [pltpu.async_copy(rows.at[j], acc.at[didx.at[j]], ssem,
                                   add=True)
                  for j in range(CH)]
            for h in sh:
                h.wait()
            return 0
        lax.fori_loop(0, rows_per_sub // CH, chunk, 0)

    @pl.when(c == 0)
    def _():
        run(ha_hbm)

    @pl.when(c == 1)
    def _():
        run(hb_hbm)

    plsc.subcore_barrier()
    pltpu.sync_copy(acc.at[pl.ds(s * SLAB, SLAB)],
                    out_hbm.at[c, pl.ds(s * SLAB, SLAB)])


@functools.partial(
    pl.kernel,
    mesh=_mesh,
    compiler_params=pltpu.CompilerParams(use_tc_tiling_on_sc=False),
    out_type=jax.ShapeDtypeStruct((2, N_PAD, OUT), jnp.float32),
    scratch_types=[
        pltpu.VMEM((CH, ROW), jnp.int32),
        pltpu.VMEM((CH, ROW), jnp.int32),
        pltpu.VMEM((CH, ROW, OUT), jnp.float32),
        pltpu.VMEM((ZR, LANE), jnp.float32),
        pltpu.VMEM_SHARED((N_PAD, OUT), jnp.float32),
        pltpu.SemaphoreType.DMA,
        pltpu.SemaphoreType.DMA,
    ],
)
def _l2_pass(src_hbm, dst_hbm, h2_hbm, out_hbm,
             sidx, didx, rows, zbuf, acc, gsem, ssem):
    c = lax.axis_index("c")
    s = lax.axis_index("s")

    _fill_zero_2d(zbuf)

    def zc(i, _):
        pltpu.sync_copy(zbuf, acc.at[pl.ds(s * SLAB + i * ZR, ZR)])
        return 0
    lax.fori_loop(0, SLAB // ZR, zc, 0)
    plsc.subcore_barrier()

    rows_per_sc = R2D // 2
    rows_per_sub = rows_per_sc // NSUB

    def chunk(k, _):
        r0 = c * rows_per_sc + s * rows_per_sub + k * CH
        pltpu.sync_copy(src_hbm.at[pl.ds(r0, CH)], sidx)
        pltpu.sync_copy(dst_hbm.at[pl.ds(r0, CH)], didx)
        gh = [pltpu.async_copy(h2_hbm.at[sidx.at[j]], rows.at[j], gsem)
              for j in range(CH)]
        for h in gh:
            h.wait()
        sh = [pltpu.async_copy(rows.at[j], acc.at[didx.at[j]], ssem, add=True)
              for j in range(CH)]
        for h in sh:
            h.wait()
        return 0
    lax.fori_loop(0, rows_per_sub // CH, chunk, 0)

    plsc.subcore_barrier()
    pltpu.sync_copy(acc.at[pl.ds(s * SLAB, SLAB)],
                    out_hbm.at[c, pl.ds(s * SLAB, SLAB)])


BN = 2000
GRID = N // BN


def _tc1_body(x_ref, degt_ref, w1_ref, ha_ref, hb_ref, dis_ref):
    degt = degt_ref[...]
    deg = degt[:, 0] + degt[:, 1] + 1.0
    dis = lax.rsqrt(deg)[:, None]
    h = jnp.dot(x_ref[...], w1_ref[...],
                preferred_element_type=jnp.float32) * dis
    ha_ref[...] = h[:, :OUT]
    hb_ref[...] = h[:, OUT:]
    dis_ref[...] = dis


def _tc1(x, degt, W1):
    return pl.pallas_call(
        _tc1_body,
        grid=(GRID,),
        in_specs=[
            pl.BlockSpec((BN, IN_CH), lambda i: (i, 0)),
            pl.BlockSpec((BN, 2), lambda i: (i, 0)),
            pl.BlockSpec((IN_CH, HID), lambda i: (0, 0)),
        ],
        out_specs=[
            pl.BlockSpec((BN, OUT), lambda i: (i, 0)),
            pl.BlockSpec((BN, OUT), lambda i: (i, 0)),
            pl.BlockSpec((BN, 1), lambda i: (i, 0)),
        ],
        out_shape=[
            jax.ShapeDtypeStruct((N, OUT), jnp.float32),
            jax.ShapeDtypeStruct((N, OUT), jnp.float32),
            jax.ShapeDtypeStruct((N, 1), jnp.float32),
        ],
    )(x, degt, W1)


def _tc2_body(oa_ref, ob_ref, ha_ref, hb_ref, dis_ref, b1_ref, w2_ref,
              out_ref):
    dis = dis_ref[...]
    o = jnp.concatenate(
        [oa_ref[...] + ha_ref[...], ob_ref[...] + hb_ref[...]], axis=1)
    z1 = jnp.maximum(o * dis + b1_ref[...], 0.0)
    h2 = jnp.dot(z1, w2_ref[...], preferred_element_type=jnp.float32)
    out_ref[...] = h2 * dis


def _tc2(oa, ob, ha, hb, dis, b1, W2):
    blk = lambda i: (i, 0)
    return pl.pallas_call(
        _tc2_body,
        grid=(GRID,),
        in_specs=[
            pl.BlockSpec((BN, OUT), blk),
            pl.BlockSpec((BN, OUT), blk),
            pl.BlockSpec((BN, OUT), blk),
            pl.BlockSpec((BN, OUT), blk),
            pl.BlockSpec((BN, 1), blk),
            pl.BlockSpec((1, HID), lambda i: (0, 0)),
            pl.BlockSpec((HID, OUT), lambda i: (0, 0)),
        ],
        out_specs=pl.BlockSpec((BN, OUT), blk),
        out_shape=jax.ShapeDtypeStruct((N, OUT), jnp.float32),
    )(oa, ob, ha, hb, dis, b1, W2)


def _tc3_body(p0_ref, p1_ref, h2_ref, dis_ref, b2_ref, z_ref):
    z_ref[...] = ((p0_ref[...] + p1_ref[...] + h2_ref[...]) * dis_ref[...]
                  + b2_ref[...])


def _tc3(p0, p1, h2, dis, b2):
    blk = lambda i: (i, 0)
    return pl.pallas_call(
        _tc3_body,
        grid=(GRID,),
        in_specs=[
            pl.BlockSpec((BN, OUT), blk),
            pl.BlockSpec((BN, OUT), blk),
            pl.BlockSpec((BN, OUT), blk),
            pl.BlockSpec((BN, 1), blk),
            pl.BlockSpec((1, OUT), lambda i: (0, 0)),
        ],
        out_specs=pl.BlockSpec((BN, OUT), blk),
        out_shape=jax.ShapeDtypeStruct((N, OUT), jnp.float32),
    )(p0, p1, h2, dis, b2)


def kernel(x, edge_index, W1, b1, W2, b2):
    src = edge_index[0].astype(jnp.int32)
    dst = edge_index[1].astype(jnp.int32)
    npad = E_PAD - E
    it = lax.iota(jnp.int32, npad)
    src_p = jnp.concatenate([src, it % NTRASH])
    dst_p = jnp.concatenate([dst, TRASH + (it % NTRASH)])
    src2d = src_p.reshape(R2D, ROW)
    dst2d = dst_p.reshape(R2D, ROW)

    degp = _deg_pass(dst2d)
    degt = jnp.transpose(degp)[:N]

    ha, hb, dis = _tc1(x, degt, W1)
    o = _l1_pass(src2d, dst2d, ha, hb)
    h2 = _tc2(o[0, :N], o[1, :N], ha, hb, dis, b1.reshape(1, HID), W2)
    p = _l2_pass(src2d, dst2d, h2)
    z = _tc3(p[0, :N], p[1, :N], h2, dis, b2.reshape(1, OUT))
    return z


# speedup vs baseline: 33.0677x; 1.0950x over previous
import functools

import jax
import jax.numpy as jnp
from jax import lax
from jax.experimental import pallas as pl
from jax.experimental.pallas import tpu as pltpu
from jax.experimental.pallas import tpu_sc as plsc

N = 100000
IN_CH = 128
HID = 32
OUT = 16

LANE = 16
ROW = 128
CH = 8
CHL = 4

NSUB = 16
N_PAD = 102400
SLAB = N_PAD // NSUB
TRASH = N
NTRASH = 1024

E = 1600000
E_PAD = 1638400
R2D = E_PAD // ROW

ZR = 100

_mesh = plsc.VectorSubcoreMesh(core_axis_name="c", subcore_axis_name="s")
_sc_params = pltpu.CompilerParams(use_tc_tiling_on_sc=False)


def _fill_zero_2d(zbuf):
    def body(i, _):
        zbuf[i, :] = jnp.zeros((LANE,), jnp.float32)
        return 0
    lax.fori_loop(0, ZR, body, 0)


def _zero_acc_2d(zbuf, acc, s):
    _fill_zero_2d(zbuf)

    def zc(i, _):
        pltpu.sync_copy(zbuf, acc.at[pl.ds(s * SLAB + i * ZR, ZR)])
        return 0
    lax.fori_loop(0, SLAB // ZR, zc, 0)
    plsc.subcore_barrier()


def _gs_pipeline(src_hbm, dst_hbm, tab_hbm, acc, sidx, didx, rows,
                 gsems, ssems, row_base, nchunks):
    def load_idx(k, b):
        r0 = row_base + k * CHL
        pltpu.sync_copy(src_hbm.at[pl.ds(r0, CHL)], sidx.at[b])
        pltpu.sync_copy(dst_hbm.at[pl.ds(r0, CHL)], didx.at[b])

    def fire_g(b):
        return [pltpu.async_copy(tab_hbm.at[sidx.at[b].at[j]],
                                 rows.at[b].at[j], gsems[b])
                for j in range(CHL)]

    def fire_s(b):
        return [pltpu.async_copy(rows.at[b].at[j], acc.at[didx.at[b].at[j]],
                                 ssems[b], add=True)
                for j in range(CHL)]

    load_idx(0, 0)
    load_idx(1, 1)

    def body(i, _):
        g0 = fire_g(0)
        g1 = fire_g(1)
        for h in g0:
            h.wait()
        s0 = fire_s(0)
        for h in g1:
            h.wait()
        s1 = fire_s(1)
        for h in s0:
            h.wait()
        load_idx(2 * i + 2, 0)
        for h in s1:
            h.wait()
        load_idx(2 * i + 3, 1)
        return 0
    lax.fori_loop(0, nchunks // 2 - 1, body, 0)

    g0 = fire_g(0)
    g1 = fire_g(1)
    for h in g0:
        h.wait()
    s0 = fire_s(0)
    for h in g1:
        h.wait()
    s1 = fire_s(1)
    for h in s0 + s1:
        h.wait()


@functools.partial(
    pl.kernel,
    mesh=_mesh,
    compiler_params=_sc_params,
    out_type=jax.ShapeDtypeStruct((2, N_PAD), jnp.float32),
    scratch_types=[
        pltpu.VMEM((2, CH, ROW), jnp.int32),
        pltpu.VMEM((ROW,), jnp.float32),
        pltpu.VMEM((SLAB,), jnp.float32),
        pltpu.VMEM_SHARED((N_PAD,), jnp.float32),
        pltpu.SemaphoreType.DMA,
        pltpu.SemaphoreType.DMA,
    ],
)
def _deg_pass(dst_hbm, out_hbm, didx, ones_v, zbuf, acc, sem0, sem1):
    c = lax.axis_index("c")
    s = lax.axis_index("s")
    w = c * NSUB + s

    for i in range(ROW // LANE):
        ones_v[pl.ds(i * LANE, LANE)] = jnp.ones((LANE,), jnp.float32)

    def fz(i, _):
        zbuf[pl.ds(i * LANE, LANE)] = jnp.zeros((LANE,), jnp.float32)
        return 0
    lax.fori_loop(0, SLAB // LANE, fz, 0)
    pltpu.sync_copy(zbuf, acc.at[pl.ds(s * SLAB, SLAB)])
    plsc.subcore_barrier()

    rows_per_w = R2D // 32
    sems = [sem0, sem1]

    def load(k, b):
        pltpu.sync_copy(dst_hbm.at[pl.ds(w * rows_per_w + k * CH, CH)],
                        didx.at[b])

    def fire(b):
        return [pltpu.async_copy(ones_v, acc.at[didx.at[b].at[j]], sems[b],
                                 add=True)
                for j in range(CH)]

    load(0, 0)
    load(1, 1)

    def chunk(i, _):
        s0 = fire(0)
        s1 = fire(1)
        for h in s0:
            h.wait()
        load(2 * i + 2, 0)
        for h in s1:
            h.wait()
        load(2 * i + 3, 1)
        return 0
    lax.fori_loop(0, rows_per_w // CH // 2 - 1, chunk, 0)

    for h in fire(0) + fire(1):
        h.wait()

    plsc.subcore_barrier()
    pltpu.sync_copy(acc.at[pl.ds(s * SLAB, SLAB)],
                    out_hbm.at[c, pl.ds(s * SLAB, SLAB)])


@functools.partial(
    pl.kernel,
    mesh=_mesh,
    compiler_params=_sc_params,
    out_type=jax.ShapeDtypeStruct((2, N_PAD, OUT), jnp.float32),
    scratch_types=[
        pltpu.VMEM((2, CHL, ROW), jnp.int32),
        pltpu.VMEM((2, CHL, ROW), jnp.int32),
        pltpu.VMEM((2, CHL, ROW, OUT), jnp.float32),
        pltpu.VMEM((ZR, LANE), jnp.float32),
        pltpu.VMEM_SHARED((N_PAD, OUT), jnp.float32),
        pltpu.SemaphoreType.DMA,
        pltpu.SemaphoreType.DMA,
        pltpu.SemaphoreType.DMA,
        pltpu.SemaphoreType.DMA,
    ],
)
def _l1_pass(src_hbm, dst_hbm, ha_hbm, hb_hbm, out_hbm,
             sidx, didx, rows, zbuf, acc, g0, g1, s0, s1):
    c = lax.axis_index("c")
    s = lax.axis_index("s")

    _zero_acc_2d(zbuf, acc, s)

    rows_per_sub = R2D // NSUB

    @pl.when(c == 0)
    def _():
        _gs_pipeline(src_hbm, dst_hbm, ha_hbm, acc, sidx, didx, rows,
                     [g0, g1], [s0, s1], s * rows_per_sub, rows_per_sub // CHL)

    @pl.when(c == 1)
    def _():
        _gs_pipeline(src_hbm, dst_hbm, hb_hbm, acc, sidx, didx, rows,
                     [g0, g1], [s0, s1], s * rows_per_sub, rows_per_sub // CHL)

    plsc.subcore_barrier()
    pltpu.sync_copy(acc.at[pl.ds(s * SLAB, SLAB)],
                    out_hbm.at[c, pl.ds(s * SLAB, SLAB)])


@functools.partial(
    pl.kernel,
    mesh=_mesh,
    compiler_params=_sc_params,
    out_type=jax.ShapeDtypeStruct((2, N_PAD, OUT), jnp.float32),
    scratch_types=[
        pltpu.VMEM((2, CHL, ROW), jnp.int32),
        pltpu.VMEM((2, CHL, ROW), jnp.int32),
        pltpu.VMEM((2, CHL, ROW, OUT), jnp.float32),
        pltpu.VMEM((ZR, LANE), jnp.float32),
        pltpu.VMEM_SHARED((N_PAD, OUT), jnp.float32),
        pltpu.SemaphoreType.DMA,
        pltpu.SemaphoreType.DMA,
        pltpu.SemaphoreType.DMA,
        pltpu.SemaphoreType.DMA,
    ],
)
def _l2_pass(src_hbm, dst_hbm, h2_hbm, out_hbm,
             sidx, didx, rows, zbuf, acc, g0, g1, s0, s1):
    c = lax.axis_index("c")
    s = lax.axis_index("s")

    _zero_acc_2d(zbuf, acc, s)

    rows_per_sc = R2D // 2
    rows_per_sub = rows_per_sc // NSUB

    _gs_pipeline(src_hbm, dst_hbm, h2_hbm, acc, sidx, didx, rows,
                 [g0, g1], [s0, s1], c * rows_per_sc + s * rows_per_sub,
                 rows_per_sub // CHL)

    plsc.subcore_barrier()
    pltpu.sync_copy(acc.at[pl.ds(s * SLAB, SLAB)],
                    out_hbm.at[c, pl.ds(s * SLAB, SLAB)])


BN = 2048
GRID = (N + BN - 1) // BN


def _tc1_body(x_ref, degp_ref, w1_ref, ha_ref, hb_ref, dis_ref):
    degt = jnp.transpose(degp_ref[...])
    deg = degt[:, :1] + degt[:, 1:2] + 1.0
    dis = lax.rsqrt(deg)
    h = jnp.dot(x_ref[...], w1_ref[...],
                preferred_element_type=jnp.float32) * dis
    ha_ref[...] = h[:, :OUT]
    hb_ref[...] = h[:, OUT:]
    dis_ref[...] = jnp.broadcast_to(dis, (BN, OUT))


def _tc1(x, degp, W1):
    return pl.pallas_call(
        _tc1_body,
        grid=(GRID,),
        in_specs=[
            pl.BlockSpec((BN, IN_CH), lambda i: (i, 0)),
            pl.BlockSpec((2, BN), lambda i: (0, i)),
            pl.BlockSpec((IN_CH, HID), lambda i: (0, 0)),
        ],
        out_specs=[
            pl.BlockSpec((BN, OUT), lambda i: (i, 0)),
            pl.BlockSpec((BN, OUT), lambda i: (i, 0)),
            pl.BlockSpec((BN, OUT), lambda i: (i, 0)),
        ],
        out_shape=[
            jax.ShapeDtypeStruct((N, OUT), jnp.float32),
            jax.ShapeDtypeStruct((N, OUT), jnp.float32),
            jax.ShapeDtypeStruct((N, OUT), jnp.float32),
        ],
    )(x, degp, W1)


def _tc2_body(oa_ref, ob_ref, ha_ref, hb_ref, dis_ref, b1_ref, w2_ref,
              out_ref):
    dis = dis_ref[...]
    o = jnp.concatenate(
        [oa_ref[0] + ha_ref[...], ob_ref[0] + hb_ref[...]], axis=1)
    dis2 = jnp.concatenate([dis, dis], axis=1)
    z1 = jnp.maximum(o * dis2 + b1_ref[...], 0.0)
    h2 = jnp.dot(z1, w2_ref[...], preferred_element_type=jnp.float32)
    out_ref[...] = h2 * dis


def _tc2(o, ha, hb, dis, b1, W2):
    blk = lambda i: (i, 0)
    return pl.pallas_call(
        _tc2_body,
        grid=(GRID,),
        in_specs=[
            pl.BlockSpec((1, BN, OUT), lambda i: (0, i, 0)),
            pl.BlockSpec((1, BN, OUT), lambda i: (1, i, 0)),
            pl.BlockSpec((BN, OUT), blk),
            pl.BlockSpec((BN, OUT), blk),
            pl.BlockSpec((BN, OUT), blk),
            pl.BlockSpec((1, HID), lambda i: (0, 0)),
            pl.BlockSpec((HID, OUT), lambda i: (0, 0)),
        ],
        out_specs=pl.BlockSpec((BN, OUT), blk),
        out_shape=jax.ShapeDtypeStruct((N, OUT), jnp.float32),
    )(o, o, ha, hb, dis, b1, W2)


def _tc3_body(p0_ref, p1_ref, h2_ref, dis_ref, b2_ref, z_ref):
    z_ref[...] = ((p0_ref[0] + p1_ref[0] + h2_ref[...]) * dis_ref[...]
                  + b2_ref[...])


def _tc3(p, h2, dis, b2):
    blk = lambda i: (i, 0)
    return pl.pallas_call(
        _tc3_body,
        grid=(GRID,),
        in_specs=[
            pl.BlockSpec((1, BN, OUT), lambda i: (0, i, 0)),
            pl.BlockSpec((1, BN, OUT), lambda i: (1, i, 0)),
            pl.BlockSpec((BN, OUT), blk),
            pl.BlockSpec((BN, OUT), blk),
            pl.BlockSpec((1, OUT), lambda i: (0, 0)),
        ],
        out_specs=pl.BlockSpec((BN, OUT), blk),
        out_shape=jax.ShapeDtypeStruct((N, OUT), jnp.float32),
    )(p, p, h2, dis, b2)


def kernel(x, edge_index, W1, b1, W2, b2):
    src = edge_index[0].astype(jnp.int32)
    dst = edge_index[1].astype(jnp.int32)
    npad = E_PAD - E
    it = lax.iota(jnp.int32, npad)
    src_p = jnp.concatenate([src, it % NTRASH])
    dst_p = jnp.concatenate([dst, TRASH + (it % NTRASH)])
    src2d = src_p.reshape(R2D, ROW)
    dst2d = dst_p.reshape(R2D, ROW)

    degp = _deg_pass(dst2d)
    ha, hb, dis = _tc1(x, degp, W1)
    o = _l1_pass(src2d, dst2d, ha, hb)
    h2 = _tc2(o, ha, hb, dis, b1.reshape(1, HID), W2)
    p = _l2_pass(src2d, dst2d, h2)
    z = _tc3(p, h2, dis, b2.reshape(1, OUT))
    return z


# speedup vs baseline: 35.0748x; 1.0607x over previous
import functools

import jax
import jax.numpy as jnp
from jax import lax
from jax.experimental import pallas as pl
from jax.experimental.pallas import tpu as pltpu
from jax.experimental.pallas import tpu_sc as plsc

N = 100000
IN_CH = 128
HID = 32
OUT = 16

LANE = 16
ROW = 128
CH = 8
CHL = 5

NSUB = 16
N_PAD = 102400
SLAB = N_PAD // NSUB
TRASH = N
NTRASH = 1024

E = 1600000
E_PAD = 1638400
R2D = E_PAD // ROW

ZR = 100

_mesh = plsc.VectorSubcoreMesh(core_axis_name="c", subcore_axis_name="s")
_sc_params = pltpu.CompilerParams(use_tc_tiling_on_sc=False)


def _fill_zero_2d(zbuf):
    def body(i, _):
        zbuf[i, :] = jnp.zeros((LANE,), jnp.float32)
        return 0
    lax.fori_loop(0, ZR, body, 0)


def _zero_acc_2d(zbuf, acc, s):
    _fill_zero_2d(zbuf)

    def zc(i, _):
        pltpu.sync_copy(zbuf, acc.at[pl.ds(s * SLAB + i * ZR, ZR)])
        return 0
    lax.fori_loop(0, SLAB // ZR, zc, 0)
    plsc.subcore_barrier()


def _gs_pipeline(src_hbm, dst_hbm, tab_hbm, acc, sidx, didx, rows,
                 gsems, ssems, row_base, nchunks):
    def load_idx(k, b):
        r0 = row_base + k * CHL
        pltpu.sync_copy(src_hbm.at[pl.ds(r0, CHL)], sidx.at[b])
        pltpu.sync_copy(dst_hbm.at[pl.ds(r0, CHL)], didx.at[b])

    def fire_g(b):
        return [pltpu.async_copy(tab_hbm.at[sidx.at[b].at[j]],
                                 rows.at[b].at[j], gsems[b])
                for j in range(CHL)]

    def fire_s(b):
        return [pltpu.async_copy(rows.at[b].at[j], acc.at[didx.at[b].at[j]],
                                 ssems[b], add=True)
                for j in range(CHL)]

    load_idx(0, 0)
    load_idx(1, 1)

    def body(i, _):
        g0 = fire_g(0)
        g1 = fire_g(1)
        for h in g0:
            h.wait()
        s0 = fire_s(0)
        for h in g1:
            h.wait()
        s1 = fire_s(1)
        for h in s0:
            h.wait()
        load_idx(2 * i + 2, 0)
        for h in s1:
            h.wait()
        load_idx(2 * i + 3, 1)
        return 0
    lax.fori_loop(0, nchunks // 2 - 1, body, 0)

    g0 = fire_g(0)
    g1 = fire_g(1)
    for h in g0:
        h.wait()
    s0 = fire_s(0)
    for h in g1:
        h.wait()
    s1 = fire_s(1)
    for h in s0 + s1:
        h.wait()


@functools.partial(
    pl.kernel,
    mesh=_mesh,
    compiler_params=_sc_params,
    out_type=jax.ShapeDtypeStruct((2, N_PAD), jnp.float32),
    scratch_types=[
        pltpu.VMEM((2, CH, ROW), jnp.int32),
        pltpu.VMEM((ROW,), jnp.float32),
        pltpu.VMEM((SLAB,), jnp.float32),
        pltpu.VMEM_SHARED((N_PAD,), jnp.float32),
        pltpu.SemaphoreType.DMA,
        pltpu.SemaphoreType.DMA,
    ],
)
def _deg_pass(dst_hbm, out_hbm, didx, ones_v, zbuf, acc, sem0, sem1):
    c = lax.axis_index("c")
    s = lax.axis_index("s")
    w = c * NSUB + s

    for i in range(ROW // LANE):
        ones_v[pl.ds(i * LANE, LANE)] = jnp.ones((LANE,), jnp.float32)

    def fz(i, _):
        zbuf[pl.ds(i * LANE, LANE)] = jnp.zeros((LANE,), jnp.float32)
        return 0
    lax.fori_loop(0, SLAB // LANE, fz, 0)
    pltpu.sync_copy(zbuf, acc.at[pl.ds(s * SLAB, SLAB)])
    plsc.subcore_barrier()

    rows_per_w = R2D // 32
    sems = [sem0, sem1]

    def load(k, b):
        pltpu.sync_copy(dst_hbm.at[pl.ds(w * rows_per_w + k * CH, CH)],
                        didx.at[b])

    def fire(b):
        return [pltpu.async_copy(ones_v, acc.at[didx.at[b].at[j]], sems[b],
                                 add=True)
                for j in range(CH)]

    load(0, 0)
    load(1, 1)

    def chunk(i, _):
        s0 = fire(0)
        s1 = fire(1)
        for h in s0:
            h.wait()
        load(2 * i + 2, 0)
        for h in s1:
            h.wait()
        load(2 * i + 3, 1)
        return 0
    lax.fori_loop(0, rows_per_w // CH // 2 - 1, chunk, 0)

    for h in fire(0) + fire(1):
        h.wait()

    plsc.subcore_barrier()
    pltpu.sync_copy(acc.at[pl.ds(s * SLAB, SLAB)],
                    out_hbm.at[c, pl.ds(s * SLAB, SLAB)])


@functools.partial(
    pl.kernel,
    mesh=_mesh,
    compiler_params=_sc_params,
    out_type=jax.ShapeDtypeStruct((2, N_PAD, OUT), jnp.float32),
    scratch_types=[
        pltpu.VMEM((2, CHL, ROW), jnp.int32),
        pltpu.VMEM((2, CHL, ROW), jnp.int32),
        pltpu.VMEM((2, CHL, ROW, OUT), jnp.float32),
        pltpu.VMEM((ZR, LANE), jnp.float32),
        pltpu.VMEM_SHARED((N_PAD, OUT), jnp.float32),
        pltpu.SemaphoreType.DMA,
        pltpu.SemaphoreType.DMA,
        pltpu.SemaphoreType.DMA,
        pltpu.SemaphoreType.DMA,
    ],
)
def _l1_pass(src_hbm, dst_hbm, ha_hbm, hb_hbm, out_hbm,
             sidx, didx, rows, zbuf, acc, g0, g1, s0, s1):
    c = lax.axis_index("c")
    s = lax.axis_index("s")

    _zero_acc_2d(zbuf, acc, s)

    rows_per_sub = R2D // NSUB

    @pl.when(c == 0)
    def _():
        _gs_pipeline(src_hbm, dst_hbm, ha_hbm, acc, sidx, didx, rows,
                     [g0, g1], [s0, s1], s * rows_per_sub, rows_per_sub // CHL)

    @pl.when(c == 1)
    def _():
        _gs_pipeline(src_hbm, dst_hbm, hb_hbm, acc, sidx, didx, rows,
                     [g0, g1], [s0, s1], s * rows_per_sub, rows_per_sub // CHL)

    plsc.subcore_barrier()
    pltpu.sync_copy(acc.at[pl.ds(s * SLAB, SLAB)],
                    out_hbm.at[c, pl.ds(s * SLAB, SLAB)])


@functools.partial(
    pl.kernel,
    mesh=_mesh,
    compiler_params=_sc_params,
    out_type=jax.ShapeDtypeStruct((2, N_PAD, OUT), jnp.float32),
    scratch_types=[
        pltpu.VMEM((2, CHL, ROW), jnp.int32),
        pltpu.VMEM((2, CHL, ROW), jnp.int32),
        pltpu.VMEM((2, CHL, ROW, OUT), jnp.float32),
        pltpu.VMEM((ZR, LANE), jnp.float32),
        pltpu.VMEM_SHARED((N_PAD, OUT), jnp.float32),
        pltpu.SemaphoreType.DMA,
        pltpu.SemaphoreType.DMA,
        pltpu.SemaphoreType.DMA,
        pltpu.SemaphoreType.DMA,
    ],
)
def _l2_pass(src_hbm, dst_hbm, h2_hbm, out_hbm,
             sidx, didx, rows, zbuf, acc, g0, g1, s0, s1):
    c = lax.axis_index("c")
    s = lax.axis_index("s")

    _zero_acc_2d(zbuf, acc, s)

    rows_per_sc = R2D // 2
    rows_per_sub = rows_per_sc // NSUB

    _gs_pipeline(src_hbm, dst_hbm, h2_hbm, acc, sidx, didx, rows,
                 [g0, g1], [s0, s1], c * rows_per_sc + s * rows_per_sub,
                 rows_per_sub // CHL)

    plsc.subcore_barrier()
    pltpu.sync_copy(acc.at[pl.ds(s * SLAB, SLAB)],
                    out_hbm.at[c, pl.ds(s * SLAB, SLAB)])


BN = 2048
GRID = (N + BN - 1) // BN


def _tc1_body(x_ref, degp_ref, w1_ref, ha_ref, hb_ref, dis_ref):
    degt = jnp.transpose(degp_ref[...])
    deg = degt[:, :1] + degt[:, 1:2] + 1.0
    dis = lax.rsqrt(deg)
    h = jnp.dot(x_ref[...], w1_ref[...],
                preferred_element_type=jnp.float32) * dis
    ha_ref[...] = h[:, :OUT]
    hb_ref[...] = h[:, OUT:]
    dis_ref[...] = jnp.broadcast_to(dis, (BN, OUT))


def _tc1(x, degp, W1):
    return pl.pallas_call(
        _tc1_body,
        grid=(GRID,),
        in_specs=[
            pl.BlockSpec((BN, IN_CH), lambda i: (i, 0)),
            pl.BlockSpec((2, BN), lambda i: (0, i)),
            pl.BlockSpec((IN_CH, HID), lambda i: (0, 0)),
        ],
        out_specs=[
            pl.BlockSpec((BN, OUT), lambda i: (i, 0)),
            pl.BlockSpec((BN, OUT), lambda i: (i, 0)),
            pl.BlockSpec((BN, OUT), lambda i: (i, 0)),
        ],
        out_shape=[
            jax.ShapeDtypeStruct((N, OUT), jnp.float32),
            jax.ShapeDtypeStruct((N, OUT), jnp.float32),
            jax.ShapeDtypeStruct((N, OUT), jnp.float32),
        ],
    )(x, degp, W1)


def _tc2_body(oa_ref, ob_ref, ha_ref, hb_ref, dis_ref, b1_ref, w2_ref,
              out_ref):
    dis = dis_ref[...]
    o = jnp.concatenate(
        [oa_ref[0] + ha_ref[...], ob_ref[0] + hb_ref[...]], axis=1)
    dis2 = jnp.concatenate([dis, dis], axis=1)
    z1 = jnp.maximum(o * dis2 + b1_ref[...], 0.0)
    h2 = jnp.dot(z1, w2_ref[...], preferred_element_type=jnp.float32)
    out_ref[...] = h2 * dis


def _tc2(o, ha, hb, dis, b1, W2):
    blk = lambda i: (i, 0)
    return pl.pallas_call(
        _tc2_body,
        grid=(GRID,),
        in_specs=[
            pl.BlockSpec((1, BN, OUT), lambda i: (0, i, 0)),
            pl.BlockSpec((1, BN, OUT), lambda i: (1, i, 0)),
            pl.BlockSpec((BN, OUT), blk),
            pl.BlockSpec((BN, OUT), blk),
            pl.BlockSpec((BN, OUT), blk),
            pl.BlockSpec((1, HID), lambda i: (0, 0)),
            pl.BlockSpec((HID, OUT), lambda i: (0, 0)),
        ],
        out_specs=pl.BlockSpec((BN, OUT), blk),
        out_shape=jax.ShapeDtypeStruct((N, OUT), jnp.float32),
    )(o, o, ha, hb, dis, b1, W2)


def _tc3_body(p0_ref, p1_ref, h2_ref, dis_ref, b2_ref, z_ref):
    z_ref[...] = ((p0_ref[0] + p1_ref[0] + h2_ref[...]) * dis_ref[...]
                  + b2_ref[...])


def _tc3(p, h2, dis, b2):
    blk = lambda i: (i, 0)
    return pl.pallas_call(
        _tc3_body,
        grid=(GRID,),
        in_specs=[
            pl.BlockSpec((1, BN, OUT), lambda i: (0, i, 0)),
            pl.BlockSpec((1, BN, OUT), lambda i: (1, i, 0)),
            pl.BlockSpec((BN, OUT), blk),
            pl.BlockSpec((BN, OUT), blk),
            pl.BlockSpec((1, OUT), lambda i: (0, 0)),
        ],
        out_specs=pl.BlockSpec((BN, OUT), blk),
        out_shape=jax.ShapeDtypeStruct((N, OUT), jnp.float32),
    )(p, p, h2, dis, b2)


def kernel(x, edge_index, W1, b1, W2, b2):
    src = edge_index[0].astype(jnp.int32)
    dst = edge_index[1].astype(jnp.int32)
    npad = E_PAD - E
    it = lax.iota(jnp.int32, npad)
    src_p = jnp.concatenate([src, it % NTRASH])
    dst_p = jnp.concatenate([dst, TRASH + (it % NTRASH)])
    src2d = src_p.reshape(R2D, ROW)
    dst2d = dst_p.reshape(R2D, ROW)

    degp = _deg_pass(dst2d)
    ha, hb, dis = _tc1(x, degp, W1)
    o = _l1_pass(src2d, dst2d, ha, hb)
    h2 = _tc2(o, ha, hb, dis, b1.reshape(1, HID), W2)
    p = _l2_pass(src2d, dst2d, h2)
    z = _tc3(p, h2, dis, b2.reshape(1, OUT))
    return z
